# baseline (speedup 1.0000x reference)
import functools

import jax
import jax.numpy as jnp
from jax.experimental import pallas as pl
from jax.experimental.pallas import tpu as pltpu

BATCH = 4
SEQ_LEN = 4096
INPUT_DIM = 2048
HIDDEN_DIM = 2048
OUTPUT_DIM = 2048

SEQ_CHUNK = 256
N_CHUNKS = SEQ_LEN // SEQ_CHUNK


def _hcen_body(x_ref, we_ref, be_ref, wo_ref, bo_ref, o_ref, acc_ref):
    i = pl.program_id(0)

    @pl.when(i == 0)
    def _init():
        acc_ref[...] = jnp.zeros_like(acc_ref)

    acc_ref[...] += jnp.sum(x_ref[...], axis=1)

    @pl.when(i == N_CHUNKS - 1)
    def _finish():
        pooled = acc_ref[...] * (1.0 / SEQ_LEN)
        dn = (((1,), (1,)), ((), ()))
        enc = jax.lax.dot_general(
            pooled, we_ref[...], dn, preferred_element_type=jnp.float32
        ) + be_ref[...]
        o_ref[...] = jax.lax.dot_general(
            enc, wo_ref[...], dn, preferred_element_type=jnp.float32
        ) + bo_ref[...]


@jax.jit
def kernel(x, W_enc, b_enc, W_out, b_out):
    b_enc2 = b_enc.reshape(1, HIDDEN_DIM)
    b_out2 = b_out.reshape(1, OUTPUT_DIM)
    return pl.pallas_call(
        _hcen_body,
        grid=(N_CHUNKS,),
        in_specs=[
            pl.BlockSpec((BATCH, SEQ_CHUNK, INPUT_DIM), lambda i: (0, i, 0)),
            pl.BlockSpec((HIDDEN_DIM, INPUT_DIM), lambda i: (0, 0)),
            pl.BlockSpec((1, HIDDEN_DIM), lambda i: (0, 0)),
            pl.BlockSpec((OUTPUT_DIM, HIDDEN_DIM), lambda i: (0, 0)),
            pl.BlockSpec((1, OUTPUT_DIM), lambda i: (0, 0)),
        ],
        out_specs=pl.BlockSpec((BATCH, OUTPUT_DIM), lambda i: (0, 0)),
        out_shape=jax.ShapeDtypeStruct((BATCH, OUTPUT_DIM), jnp.float32),
        scratch_shapes=[pltpu.VMEM((BATCH, INPUT_DIM), jnp.float32)],
    )(x, W_enc, b_enc2, W_out, b_out2)
